# baseline (speedup 1.0000x reference)
import functools

import jax
import jax.numpy as jnp
from jax import lax
from jax.experimental import pallas as pl
from jax.experimental.pallas import tpu as pltpu
from jax.experimental.pallas import tpu_sc as plsc

BATCH = 16384
FEAT = 64
NCLASS = 100000
P0 = 33280
P1 = 66560
PMAIN = 99968 - P1
CBUF = PMAIN + 128
LANES = 16
NC = 2
NS = 16
NW = NC * NS
RPW = FEAT // NW
NPIECE = 3
NPASS = RPW * NPIECE
UNROLL = 16
NITER = BATCH // (LANES * UNROLL)

_LO = (0, P0, P1)
_LEN = (P0, P1 - P0, PMAIN)


def _make_kernel():
    mesh = plsc.VectorSubcoreMesh(core_axis_name="c", subcore_axis_name="s")

    @functools.partial(
        pl.kernel,
        mesh=mesh,
        compiler_params=pltpu.CompilerParams(needs_layout_passes=False),
        out_type=jax.ShapeDtypeStruct((NW, LANES), jnp.float32),
        scratch_types=[
            pltpu.VMEM((CBUF,), jnp.float32),
            pltpu.VMEM((CBUF,), jnp.float32),
            pltpu.VMEM((BATCH,), jnp.int32),
            pltpu.VMEM((BATCH,), jnp.float32),
            pltpu.VMEM((BATCH,), jnp.float32),
            pltpu.VMEM((LANES,), jnp.float32),
            pltpu.SemaphoreType.DMA,
            pltpu.SemaphoreType.DMA,
            pltpu.SemaphoreType.DMA,
            pltpu.SemaphoreType.DMA,
            pltpu.SemaphoreType.DMA,
        ],
    )
    def _k(xt_hbm, labels_hbm, ct_hbm, tail_hbm, out_hbm, c0_v, c1_v,
           labels_v, x0_v, x1_v, acc_v, sc0, sc1, sx0, sx1, sl):
        wid = lax.axis_index("s") * NC + lax.axis_index("c")
        cbufs, csems = (c0_v, c1_v), (sc0, sc1)
        xbufs, xsems = (x0_v, x1_v), (sx0, sx1)

        def f_of(q):
            return wid * RPW + q // NPIECE

        def c_fire(q):
            p = q % NPIECE
            buf, sem = cbufs[q % 2], csems[q % 2]
            f = f_of(q)
            if p < NPIECE - 1:
                return (pltpu.async_copy(
                    ct_hbm.at[f, pl.ds(_LO[p], _LEN[p])],
                    buf.at[pl.ds(0, _LEN[p])], sem),)
            return (
                pltpu.async_copy(
                    ct_hbm.at[f, pl.ds(P1, PMAIN)],
                    buf.at[pl.ds(0, PMAIN)], sem),
                pltpu.async_copy(
                    tail_hbm.at[f], buf.at[pl.ds(PMAIN, 128)], sem),
            )

        def x_fire(r):
            return pltpu.async_copy(
                xt_hbm.at[wid * RPW + r], xbufs[r % 2], xsems[r % 2])

        cp_l = pltpu.async_copy(labels_hbm, labels_v, sl)
        c_pend = c_fire(0)
        x_pend = x_fire(0)
        cp_l.wait()

        accs = tuple(jnp.zeros((LANES,), jnp.float32) for _ in range(UNROLL))
        for q in range(NPASS):
            p = q % NPIECE
            if q + 1 < NPASS:
                c_next = c_fire(q + 1)
            if p == 0:
                x_pend.wait()
                if q // NPIECE + 1 < RPW:
                    x_nextrow = x_fire(q // NPIECE + 1)
            for cp in c_pend:
                cp.wait()
            cref = cbufs[q % 2]
            xbuf = xbufs[(q // NPIECE) % 2]

            @plsc.parallel_loop(0, NITER, carry=accs)
            def accs(i, accs, p=p, xbuf=xbuf, cref=cref):
                outs = []
                for u in range(UNROLL):
                    o = i * (LANES * UNROLL) + u * LANES
                    idx = labels_v[pl.ds(o, LANES)]
                    xv = xbuf[pl.ds(o, LANES)]
                    if p == 0:
                        mask = idx < P0
                        rel = idx
                    elif p == 1:
                        rel = idx - P0
                        mask = rel.astype(jnp.uint32) < jnp.uint32(P1 - P0)
                    else:
                        mask = idx >= P1
                        rel = idx - P1
                    c = plsc.load_gather(cref, [rel], mask=mask)
                    d = jnp.where(mask, xv - c, 0.0)
                    outs.append(accs[u] + d * d)
                return tuple(outs)

            if q + 1 < NPASS:
                c_pend = c_next
            if p == 0 and q // NPIECE + 1 < RPW:
                x_pend = x_nextrow
        acc_v[...] = sum(accs[1:], accs[0]) * 0.5
        pltpu.sync_copy(acc_v, out_hbm.at[wid])

    return _k


_sc_kernel = _make_kernel()


def kernel(x, labels, centers):
    tail = jnp.zeros((FEAT, 128), jnp.float32)
    tail = lax.dynamic_update_slice(tail, centers[P1 + PMAIN:].T, (0, 0))
    partials = _sc_kernel(x.T, labels.astype(jnp.int32), centers.T, tail)
    return jnp.sum(partials)


# speedup vs baseline: 1.0291x; 1.0291x over previous
import functools

import jax
import jax.numpy as jnp
from jax import lax
from jax.experimental import pallas as pl
from jax.experimental.pallas import tpu as pltpu
from jax.experimental.pallas import tpu_sc as plsc

BATCH = 16384
FEAT = 64
NCLASS = 100000
P0 = 33280
P1 = 66560
PMAIN = 99968 - P1
CBUF = PMAIN + 128
LANES = 16
NC = 2
NS = 16
NW = NC * NS
RPW = FEAT // NW
NPIECE = 3
NPASS = RPW * NPIECE
UNROLL = 8
NITER = BATCH // (LANES * UNROLL)

_LO = (0, P0, P1)
_LEN = (P0, P1 - P0, PMAIN)


def _make_kernel():
    mesh = plsc.VectorSubcoreMesh(core_axis_name="c", subcore_axis_name="s")

    @functools.partial(
        pl.kernel,
        mesh=mesh,
        compiler_params=pltpu.CompilerParams(needs_layout_passes=False),
        out_type=jax.ShapeDtypeStruct((NW, LANES), jnp.float32),
        scratch_types=[
            pltpu.VMEM((CBUF,), jnp.float32),
            pltpu.VMEM((CBUF,), jnp.float32),
            pltpu.VMEM((BATCH,), jnp.int32),
            pltpu.VMEM((BATCH,), jnp.float32),
            pltpu.VMEM((BATCH,), jnp.float32),
            pltpu.VMEM((LANES,), jnp.float32),
            pltpu.SemaphoreType.DMA,
            pltpu.SemaphoreType.DMA,
            pltpu.SemaphoreType.DMA,
            pltpu.SemaphoreType.DMA,
            pltpu.SemaphoreType.DMA,
        ],
    )
    def _k(xt_hbm, labels_hbm, ct_hbm, tail_hbm, out_hbm, c0_v, c1_v,
           labels_v, x0_v, x1_v, acc_v, sc0, sc1, sx0, sx1, sl):
        wid = lax.axis_index("s") * NC + lax.axis_index("c")
        cbufs, csems = (c0_v, c1_v), (sc0, sc1)
        xbufs, xsems = (x0_v, x1_v), (sx0, sx1)

        def f_of(q):
            return wid * RPW + q // NPIECE

        def c_fire(q):
            p = q % NPIECE
            buf, sem = cbufs[q % 2], csems[q % 2]
            f = f_of(q)
            if p < NPIECE - 1:
                return (pltpu.async_copy(
                    ct_hbm.at[f, pl.ds(_LO[p], _LEN[p])],
                    buf.at[pl.ds(0, _LEN[p])], sem),)
            return (
                pltpu.async_copy(
                    ct_hbm.at[f, pl.ds(P1, PMAIN)],
                    buf.at[pl.ds(0, PMAIN)], sem),
                pltpu.async_copy(
                    tail_hbm.at[f], buf.at[pl.ds(PMAIN, 128)], sem),
            )

        def x_fire(r):
            return pltpu.async_copy(
                xt_hbm.at[wid * RPW + r], xbufs[r % 2], xsems[r % 2])

        cp_l = pltpu.async_copy(labels_hbm, labels_v, sl)
        c_pend = c_fire(0)
        x_pend = x_fire(0)
        cp_l.wait()

        accs = tuple(jnp.zeros((LANES,), jnp.float32) for _ in range(UNROLL))
        for q in range(NPASS):
            p = q % NPIECE
            if q + 1 < NPASS:
                c_next = c_fire(q + 1)
            if p == 0:
                x_pend.wait()
                if q // NPIECE + 1 < RPW:
                    x_nextrow = x_fire(q // NPIECE + 1)
            for cp in c_pend:
                cp.wait()
            cref = cbufs[q % 2]
            xbuf = xbufs[(q // NPIECE) % 2]

            @plsc.parallel_loop(0, NITER, carry=accs)
            def accs(i, accs, p=p, xbuf=xbuf, cref=cref):
                outs = []
                for u in range(UNROLL):
                    o = i * (LANES * UNROLL) + u * LANES
                    idx = labels_v[pl.ds(o, LANES)]
                    xv = xbuf[pl.ds(o, LANES)]
                    if p == 0:
                        mask = idx < P0
                        rel = idx
                    elif p == 1:
                        rel = idx - P0
                        mask = rel.astype(jnp.uint32) < jnp.uint32(P1 - P0)
                    else:
                        mask = idx >= P1
                        rel = idx - P1
                    c = plsc.load_gather(cref, [rel], mask=mask)
                    d = jnp.where(mask, xv - c, 0.0)
                    outs.append(accs[u] + d * d)
                return tuple(outs)

            if q + 1 < NPASS:
                c_pend = c_next
            if p == 0 and q // NPIECE + 1 < RPW:
                x_pend = x_nextrow
        acc_v[...] = sum(accs[1:], accs[0]) * 0.5
        pltpu.sync_copy(acc_v, out_hbm.at[wid])

    return _k


_sc_kernel = _make_kernel()


def kernel(x, labels, centers):
    tail = jnp.zeros((FEAT, 128), jnp.float32)
    tail = lax.dynamic_update_slice(tail, centers[P1 + PMAIN:].T, (0, 0))
    partials = _sc_kernel(x.T, labels.astype(jnp.int32), centers.T, tail)
    return jnp.sum(partials)


# speedup vs baseline: 1.0321x; 1.0029x over previous
import functools

import jax
import jax.numpy as jnp
from jax import lax
from jax.experimental import pallas as pl
from jax.experimental.pallas import tpu as pltpu
from jax.experimental.pallas import tpu_sc as plsc

BATCH = 16384
FEAT = 64
NCLASS = 100000
P0 = 33280
P1 = 66560
PMAIN = 99968 - P1
CBUF = PMAIN + 128
LANES = 16
NC = 2
NS = 16
NW = NC * NS
RPW = FEAT // NW
NPIECE = 3
NPASS = RPW * NPIECE
UNROLL = 8
NITER = BATCH // (LANES * UNROLL)

_LO = (0, P0, P1)
_LEN = (P0, P1 - P0, PMAIN)


def _make_kernel():
    mesh = plsc.VectorSubcoreMesh(core_axis_name="c", subcore_axis_name="s")

    @functools.partial(
        pl.kernel,
        mesh=mesh,
        compiler_params=pltpu.CompilerParams(needs_layout_passes=False),
        out_type=jax.ShapeDtypeStruct((NW, LANES), jnp.float32),
        scratch_types=[
            pltpu.VMEM((CBUF,), jnp.float32),
            pltpu.VMEM((CBUF,), jnp.float32),
            pltpu.VMEM((BATCH,), jnp.int32),
            pltpu.VMEM((BATCH,), jnp.float32),
            pltpu.VMEM((BATCH,), jnp.float32),
            pltpu.VMEM((LANES,), jnp.float32),
            pltpu.SemaphoreType.DMA,
            pltpu.SemaphoreType.DMA,
            pltpu.SemaphoreType.DMA,
            pltpu.SemaphoreType.DMA,
            pltpu.SemaphoreType.DMA,
        ],
    )
    def _k(xt_hbm, labels_hbm, ct_hbm, tail_hbm, out_hbm, c0_v, c1_v,
           labels_v, x0_v, x1_v, acc_v, sc0, sc1, sx0, sx1, sl):
        wid = lax.axis_index("c") * NS + lax.axis_index("s")
        cbufs, csems = (c0_v, c1_v), (sc0, sc1)
        xbufs, xsems = (x0_v, x1_v), (sx0, sx1)

        def f_of(q):
            return wid * RPW + q // NPIECE

        def c_fire(q):
            p = q % NPIECE
            buf, sem = cbufs[q % 2], csems[q % 2]
            f = f_of(q)
            if p < NPIECE - 1:
                return (pltpu.async_copy(
                    ct_hbm.at[f, pl.ds(_LO[p], _LEN[p])],
                    buf.at[pl.ds(0, _LEN[p])], sem),)
            return (
                pltpu.async_copy(
                    ct_hbm.at[f, pl.ds(P1, PMAIN)],
                    buf.at[pl.ds(0, PMAIN)], sem),
                pltpu.async_copy(
                    tail_hbm.at[f], buf.at[pl.ds(PMAIN, 128)], sem),
            )

        def x_fire(r):
            return pltpu.async_copy(
                xt_hbm.at[wid * RPW + r], xbufs[r % 2], xsems[r % 2])

        cp_l = pltpu.async_copy(labels_hbm, labels_v, sl)
        c_pend = c_fire(0)
        x_pend = x_fire(0)
        cp_l.wait()

        accs = tuple(jnp.zeros((LANES,), jnp.float32) for _ in range(UNROLL))
        for q in range(NPASS):
            p = q % NPIECE
            if q + 1 < NPASS:
                c_next = c_fire(q + 1)
            if p == 0:
                x_pend.wait()
                if q // NPIECE + 1 < RPW:
                    x_nextrow = x_fire(q // NPIECE + 1)
            for cp in c_pend:
                cp.wait()
            cref = cbufs[q % 2]
            xbuf = xbufs[(q // NPIECE) % 2]

            @plsc.parallel_loop(0, NITER, carry=accs)
            def accs(i, accs, p=p, xbuf=xbuf, cref=cref):
                outs = []
                for u in range(UNROLL):
                    o = i * (LANES * UNROLL) + u * LANES
                    idx = labels_v[pl.ds(o, LANES)]
                    xv = xbuf[pl.ds(o, LANES)]
                    if p == 0:
                        mask = idx < P0
                        rel = idx
                    elif p == 1:
                        rel = idx - P0
                        mask = rel.astype(jnp.uint32) < jnp.uint32(P1 - P0)
                    else:
                        mask = idx >= P1
                        rel = idx - P1
                    c = plsc.load_gather(cref, [rel], mask=mask)
                    d = jnp.where(mask, xv - c, 0.0)
                    outs.append(accs[u] + d * d)
                return tuple(outs)

            if q + 1 < NPASS:
                c_pend = c_next
            if p == 0 and q // NPIECE + 1 < RPW:
                x_pend = x_nextrow
        acc_v[...] = sum(accs[1:], accs[0]) * 0.5
        pltpu.sync_copy(acc_v, out_hbm.at[wid])

    return _k


_sc_kernel = _make_kernel()


def kernel(x, labels, centers):
    tail = jnp.zeros((FEAT, 128), jnp.float32)
    tail = lax.dynamic_update_slice(tail, centers[P1 + PMAIN:].T, (0, 0))
    partials = _sc_kernel(x.T, labels.astype(jnp.int32), centers.T, tail)
    return jnp.sum(partials)
